# baseline (speedup 1.0000x reference)
import functools

import jax
import jax.numpy as jnp
from jax import lax
from jax.experimental import pallas as pl
from jax.experimental.pallas import tpu as pltpu
from jax.experimental.pallas import tpu_sc as plsc

MAX_LEN = 8192
EMBED_DIM = 1024
BATCH = 4
SEQ = 8192

_B_TOTAL = BATCH * SEQ
_NC = 2
_NS = 16
_NW = _NC * _NS
_B_PER_W = _B_TOTAL // _NW
_CH = 32
_NCH = _B_PER_W // _CH
_NBUF = 2
_NGRP = _NCH // _NBUF


@functools.partial(
    pl.kernel,
    out_type=jax.ShapeDtypeStruct((_B_TOTAL, EMBED_DIM), jnp.float32),
    mesh=plsc.VectorSubcoreMesh(core_axis_name="c", subcore_axis_name="s"),
    scratch_types=[
        pltpu.VMEM((_B_PER_W,), jnp.int32),
        pltpu.VMEM((_NBUF, _CH, EMBED_DIM), jnp.float32),
        pltpu.SemaphoreType.DMA,
        pltpu.SemaphoreType.DMA,
        pltpu.SemaphoreType.DMA,
        pltpu.SemaphoreType.DMA,
    ],
)
def _gather_rows(idx_hbm, table_hbm, out_hbm, idx_v, rows_v, g0, g1, w0, w1):
    gsem = (g0, g1)
    wsem = (w0, w1)
    wid = lax.axis_index("s") * _NC + lax.axis_index("c")
    base = wid * _B_PER_W

    pltpu.sync_copy(idx_hbm.at[pl.ds(base, _B_PER_W)], idx_v)

    def start_gather(c, b):
        pltpu.async_copy(
            table_hbm.at[idx_v.at[pl.ds(c * _CH, _CH)]], rows_v.at[b], gsem[b])

    def wait_gather(c, b):
        pltpu.make_async_copy(
            table_hbm.at[idx_v.at[pl.ds(c * _CH, _CH)]], rows_v.at[b],
            gsem[b]).wait()

    def start_wb(c, b):
        off = base + c * _CH
        pltpu.async_copy(rows_v.at[b], out_hbm.at[pl.ds(off, _CH)], wsem[b])

    def wait_wb(b):
        pltpu.make_async_copy(
            rows_v.at[b], out_hbm.at[pl.ds(base, _CH)], wsem[b]).wait()

    start_gather(0, 0)
    wait_gather(0, 0)
    start_wb(0, 0)
    start_gather(1, 1)

    def pair_body(g, _):
        for j in range(2):
            c = 1 + g * 2 + j
            b = (1 + j) % 2
            wait_gather(c, b)
            start_wb(c, b)
            wait_wb(1 - b)
            start_gather(c + 1, 1 - b)
        return 0

    lax.fori_loop(0, (_NCH - 2) // 2, pair_body, 0)

    wait_gather(_NCH - 1, 1)
    start_wb(_NCH - 1, 1)
    wait_wb(0)
    wait_wb(1)


def kernel(input_pos_tensors, table):
    idx_flat = input_pos_tensors.reshape(-1).astype(jnp.int32)
    out = _gather_rows(idx_flat, table)
    return out.reshape(BATCH, SEQ, EMBED_DIM)


# speedup vs baseline: 1.0228x; 1.0228x over previous
import functools

import jax
import jax.numpy as jnp
from jax import lax
from jax.experimental import pallas as pl
from jax.experimental.pallas import tpu as pltpu
from jax.experimental.pallas import tpu_sc as plsc

MAX_LEN = 8192
EMBED_DIM = 1024
BATCH = 4
SEQ = 8192

_B_TOTAL = BATCH * SEQ
_NC = 2
_NS = 16
_NW = _NC * _NS
_B_PER_W = _B_TOTAL // _NW
_CH = 32
_NCH = _B_PER_W // _CH
_NBUF = 3


@functools.partial(
    pl.kernel,
    out_type=jax.ShapeDtypeStruct((_B_TOTAL, EMBED_DIM), jnp.float32),
    mesh=plsc.VectorSubcoreMesh(core_axis_name="c", subcore_axis_name="s"),
    scratch_types=[
        pltpu.VMEM((_B_PER_W,), jnp.int32),
        pltpu.VMEM((_NBUF, _CH, EMBED_DIM), jnp.float32),
        pltpu.SemaphoreType.DMA,
        pltpu.SemaphoreType.DMA,
        pltpu.SemaphoreType.DMA,
        pltpu.SemaphoreType.DMA,
        pltpu.SemaphoreType.DMA,
        pltpu.SemaphoreType.DMA,
    ],
)
def _gather_rows(idx_hbm, table_hbm, out_hbm, idx_v, rows_v,
                 g0, g1, g2, w0, w1, w2):
    gsem = (g0, g1, g2)
    wsem = (w0, w1, w2)
    wid = lax.axis_index("s") * _NC + lax.axis_index("c")
    base = wid * _B_PER_W

    pltpu.sync_copy(idx_hbm.at[pl.ds(base, _B_PER_W)], idx_v)

    def start_gather(c, b):
        pltpu.async_copy(
            table_hbm.at[idx_v.at[pl.ds(c * _CH, _CH)]], rows_v.at[b], gsem[b])

    def wait_gather(c, b):
        pltpu.make_async_copy(
            table_hbm.at[idx_v.at[pl.ds(c * _CH, _CH)]], rows_v.at[b],
            gsem[b]).wait()

    def start_wb(c, b):
        off = base + c * _CH
        pltpu.async_copy(rows_v.at[b], out_hbm.at[pl.ds(off, _CH)], wsem[b])

    def wait_wb(b):
        pltpu.make_async_copy(
            rows_v.at[b], out_hbm.at[pl.ds(base, _CH)], wsem[b]).wait()

    start_gather(0, 0)
    start_gather(1, 1)
    wait_gather(0, 0)
    start_wb(0, 0)
    start_gather(2, 2)

    def trio_body(g, _):
        for j in range(3):
            c = 1 + g * 3 + j
            b = (1 + j) % 3
            wait_gather(c, b)
            start_wb(c, b)
            wait_wb((b + 2) % 3)
            start_gather(c + 2, (b + 2) % 3)
        return 0

    lax.fori_loop(0, (_NCH - 5) // 3, trio_body, 0)

    for c in range(_NCH - 4, _NCH):
        b = c % 3
        wait_gather(c, b)
        start_wb(c, b)
        wait_wb((b + 2) % 3)
        if c + 2 < _NCH:
            start_gather(c + 2, (b + 2) % 3)
    wait_wb((_NCH - 1) % 3)


def kernel(input_pos_tensors, table):
    idx_flat = input_pos_tensors.reshape(-1).astype(jnp.int32)
    out = _gather_rows(idx_flat, table)
    return out.reshape(BATCH, SEQ, EMBED_DIM)


# speedup vs baseline: 1.0347x; 1.0116x over previous
import functools

import jax
import jax.numpy as jnp
from jax import lax
from jax.experimental import pallas as pl
from jax.experimental.pallas import tpu as pltpu
from jax.experimental.pallas import tpu_sc as plsc

MAX_LEN = 8192
EMBED_DIM = 1024
BATCH = 4
SEQ = 8192

_B_TOTAL = BATCH * SEQ
_NC = 2
_NS = 16
_NW = _NC * _NS
_B_PER_W = _B_TOTAL // _NW
_CH = 16
_NCH = _B_PER_W // _CH
_NBUF = 6


@functools.partial(
    pl.kernel,
    out_type=jax.ShapeDtypeStruct((_B_TOTAL, EMBED_DIM), jnp.float32),
    mesh=plsc.VectorSubcoreMesh(core_axis_name="c", subcore_axis_name="s"),
    scratch_types=[
        pltpu.VMEM((_B_PER_W,), jnp.int32),
        pltpu.VMEM((_NBUF, _CH, EMBED_DIM), jnp.float32),
        pltpu.SemaphoreType.DMA,
        pltpu.SemaphoreType.DMA,
        pltpu.SemaphoreType.DMA,
        pltpu.SemaphoreType.DMA,
        pltpu.SemaphoreType.DMA,
        pltpu.SemaphoreType.DMA,
        pltpu.SemaphoreType.DMA,
        pltpu.SemaphoreType.DMA,
        pltpu.SemaphoreType.DMA,
        pltpu.SemaphoreType.DMA,
        pltpu.SemaphoreType.DMA,
        pltpu.SemaphoreType.DMA,
    ],
)
def _gather_rows(idx_hbm, table_hbm, out_hbm, idx_v, rows_v,
                 g0, g1, g2, g3, g4, g5, w0, w1, w2, w3, w4, w5):
    gsem = (g0, g1, g2, g3, g4, g5)
    wsem = (w0, w1, w2, w3, w4, w5)
    wid = lax.axis_index("s") * _NC + lax.axis_index("c")
    base = wid * _B_PER_W

    pltpu.sync_copy(idx_hbm.at[pl.ds(base, _B_PER_W)], idx_v)

    def start_gather(c, b):
        pltpu.async_copy(
            table_hbm.at[idx_v.at[pl.ds(c * _CH, _CH)]], rows_v.at[b], gsem[b])

    def wait_gather(c, b):
        pltpu.make_async_copy(
            table_hbm.at[idx_v.at[pl.ds(c * _CH, _CH)]], rows_v.at[b],
            gsem[b]).wait()

    def start_wb(c, b):
        off = base + c * _CH
        pltpu.async_copy(rows_v.at[b], out_hbm.at[pl.ds(off, _CH)], wsem[b])

    def wait_wb(b):
        pltpu.make_async_copy(
            rows_v.at[b], out_hbm.at[pl.ds(base, _CH)], wsem[b]).wait()

    for c in range(3):
        start_gather(c, c)
    for c in range(3):
        wait_gather(c, c)
        start_wb(c, c)
        start_gather(c + 3, c + 3)

    def hex_body(g, _):
        for j in range(6):
            c = 3 + g * 6 + j
            b = (3 + j) % 6
            wait_gather(c, b)
            start_wb(c, b)
            wait_wb((b + 3) % 6)
            start_gather(c + 3, (b + 3) % 6)
        return 0

    lax.fori_loop(0, (_NCH - 10) // 6, hex_body, 0)

    for c in range(_NCH - 7, _NCH):
        b = c % 6
        wait_gather(c, b)
        start_wb(c, b)
        wait_wb((b + 3) % 6)
        if c + 3 < _NCH:
            start_gather(c + 3, (b + 3) % 6)
    for c in range(_NCH - 3, _NCH):
        wait_wb(c % 6)


def kernel(input_pos_tensors, table):
    idx_flat = input_pos_tensors.reshape(-1).astype(jnp.int32)
    out = _gather_rows(idx_flat, table)
    return out.reshape(BATCH, SEQ, EMBED_DIM)


# speedup vs baseline: 1.0356x; 1.0008x over previous
import functools

import jax
import jax.numpy as jnp
from jax import lax
from jax.experimental import pallas as pl
from jax.experimental.pallas import tpu as pltpu
from jax.experimental.pallas import tpu_sc as plsc

MAX_LEN = 8192
EMBED_DIM = 1024
BATCH = 4
SEQ = 8192

_B_TOTAL = BATCH * SEQ
_NC = 2
_NS = 16
_NW = _NC * _NS
_B_PER_W = _B_TOTAL // _NW
_CH = 16
_NCH = _B_PER_W // _CH
_NBUF = 6


@functools.partial(
    pl.kernel,
    out_type=jax.ShapeDtypeStruct((_B_TOTAL, EMBED_DIM), jnp.float32),
    mesh=plsc.VectorSubcoreMesh(core_axis_name="c", subcore_axis_name="s"),
    scratch_types=[
        pltpu.VMEM((_B_PER_W,), jnp.int32),
        pltpu.VMEM((_NBUF, _CH, EMBED_DIM), jnp.float32),
        pltpu.SemaphoreType.DMA,
        pltpu.SemaphoreType.DMA,
        pltpu.SemaphoreType.DMA,
        pltpu.SemaphoreType.DMA,
        pltpu.SemaphoreType.DMA,
        pltpu.SemaphoreType.DMA,
        pltpu.SemaphoreType.DMA,
        pltpu.SemaphoreType.DMA,
        pltpu.SemaphoreType.DMA,
        pltpu.SemaphoreType.DMA,
        pltpu.SemaphoreType.DMA,
        pltpu.SemaphoreType.DMA,
    ],
)
def _gather_rows(idx_hbm, table_hbm, out_hbm, idx_v, rows_v,
                 g0, g1, g2, g3, g4, g5, w0, w1, w2, w3, w4, w5):
    gsem = (g0, g1, g2, g3, g4, g5)
    wsem = (w0, w1, w2, w3, w4, w5)
    wid = lax.axis_index("s") * _NC + lax.axis_index("c")
    base = wid * _B_PER_W

    pltpu.sync_copy(idx_hbm.at[pl.ds(base, _B_PER_W)], idx_v)

    def start_gather(c, b):
        pltpu.async_copy(
            table_hbm.at[idx_v.at[pl.ds(c * _CH, _CH)]], rows_v.at[b], gsem[b])

    def wait_gather(c, b):
        pltpu.make_async_copy(
            table_hbm.at[idx_v.at[pl.ds(c * _CH, _CH)]], rows_v.at[b],
            gsem[b]).wait()

    def start_wb(c, b):
        off = base + c * _CH
        pltpu.async_copy(rows_v.at[b], out_hbm.at[pl.ds(off, _CH)], wsem[b])

    def wait_wb(b):
        pltpu.make_async_copy(
            rows_v.at[b], out_hbm.at[pl.ds(base, _CH)], wsem[b]).wait()

    for c in range(4):
        start_gather(c, c)
    for c in range(2):
        wait_gather(c, c)
        start_wb(c, c)
        start_gather(c + 4, c + 4)

    def hex_body(g, _):
        for j in range(6):
            c = 2 + g * 6 + j
            b = (2 + j) % 6
            wait_gather(c, b)
            start_wb(c, b)
            wait_wb((b + 4) % 6)
            start_gather(c + 4, (b + 4) % 6)
        return 0

    lax.fori_loop(0, (_NCH - 8) // 6, hex_body, 0)

    for c in range(_NCH - 8, _NCH):
        b = c % 6
        wait_gather(c, b)
        start_wb(c, b)
        wait_wb((b + 4) % 6)
        if c + 4 < _NCH:
            start_gather(c + 4, (b + 4) % 6)
    for c in range(_NCH - 2, _NCH):
        wait_wb(c % 6)


def kernel(input_pos_tensors, table):
    idx_flat = input_pos_tensors.reshape(-1).astype(jnp.int32)
    out = _gather_rows(idx_flat, table)
    return out.reshape(BATCH, SEQ, EMBED_DIM)
